# Initial kernel scaffold; baseline (speedup 1.0000x reference)
import jax
import jax.numpy as jnp
from jax.experimental import pallas as pl


def kernel(image, label):
    raise NotImplementedError("write your pallas kernel here")


# speedup vs baseline: 868.7563x; 868.7563x over previous
import functools

import jax
import jax.numpy as jnp
from jax import lax
from jax.experimental import pallas as pl
from jax.experimental.pallas import tpu as pltpu
from jax.experimental.pallas import tpu_sc as plsc

NC = 2
NS = 16
NW = NC * NS
L = 16
NVALS = 256
N = 16 * 512 * 512
PER_W = N // NW
CHUNK = 8192
NCHUNK = PER_W // CHUNK

_mesh = plsc.VectorSubcoreMesh(core_axis_name="c", subcore_axis_name="s")
_params = pltpu.CompilerParams(needs_layout_passes=False)


def _presence_body(label_hbm, out_hbm, buf, table):
    c = lax.axis_index("c")
    s = lax.axis_index("s")
    w = s * NC + c
    zeros = jnp.zeros((L,), jnp.int32)
    for i in range(NVALS // L):
        table[pl.ds(i * L, L)] = zeros
    ones = jnp.ones((L,), jnp.int32)
    base = w * jnp.int32(PER_W)

    def chunk_body(k, _):
        pltpu.sync_copy(
            label_hbm.at[pl.ds(base + k * jnp.int32(CHUNK), CHUNK)], buf)

        def inner(i, _):
            idx = buf[pl.ds(i * jnp.int32(L), L)]
            plsc.store_scatter(table, [idx], ones)
            return 0

        lax.fori_loop(jnp.int32(0), jnp.int32(CHUNK // L), inner, 0)
        return 0

    lax.fori_loop(jnp.int32(0), jnp.int32(NCHUNK), chunk_body, 0)
    pltpu.sync_copy(table, out_hbm.at[w])


_presence_call = functools.partial(
    pl.kernel,
    out_type=jax.ShapeDtypeStruct((NW, NVALS), jnp.int32),
    mesh=_mesh,
    compiler_params=_params,
    scratch_types=[
        pltpu.VMEM((CHUNK,), jnp.int32),
        pltpu.VMEM((NVALS,), jnp.int32),
    ],
)(_presence_body)


def _lookup_body(label_hbm, pres_hbm, out_hbm, pbuf, table, lbuf, obuf):
    c = lax.axis_index("c")
    s = lax.axis_index("s")
    w = s * NC + c
    pltpu.sync_copy(pres_hbm, pbuf)
    carry = jnp.int32(0)
    for ci in range(NVALS // L):
        acc = jnp.zeros((L,), jnp.int32)
        for r in range(NW):
            acc = acc + pbuf[r, pl.ds(ci * L, L)]
        ind = (acc > 0).astype(jnp.int32)
        incl = plsc.cumsum(ind)
        table[pl.ds(ci * L, L)] = (incl - ind) + carry
        carry = carry + jnp.sum(ind, dtype=jnp.int32)

    base = w * jnp.int32(PER_W)

    def chunk_body(k, _):
        off = base + k * jnp.int32(CHUNK)
        pltpu.sync_copy(label_hbm.at[pl.ds(off, CHUNK)], lbuf)

        def inner(i, _):
            sl = pl.ds(i * jnp.int32(L), L)
            obuf[sl] = plsc.load_gather(table, [lbuf[sl]])
            return 0

        lax.fori_loop(jnp.int32(0), jnp.int32(CHUNK // L), inner, 0)
        pltpu.sync_copy(obuf, out_hbm.at[pl.ds(off, CHUNK)])
        return 0

    lax.fori_loop(jnp.int32(0), jnp.int32(NCHUNK), chunk_body, 0)


_lookup_call = functools.partial(
    pl.kernel,
    out_type=jax.ShapeDtypeStruct((N,), jnp.int32),
    mesh=_mesh,
    compiler_params=_params,
    scratch_types=[
        pltpu.VMEM((NW, NVALS), jnp.int32),
        pltpu.VMEM((NVALS,), jnp.int32),
        pltpu.VMEM((CHUNK,), jnp.int32),
        pltpu.VMEM((CHUNK,), jnp.int32),
    ],
)(_lookup_body)


def kernel(image, label):
    flat = label.reshape(-1).astype(jnp.int32)
    pres = _presence_call(flat)
    out32 = _lookup_call(flat, pres)
    new_label = out32.astype(label.dtype).reshape(label.shape)
    return (image, new_label)


# speedup vs baseline: 933.5457x; 1.0746x over previous
import functools

import jax
import jax.numpy as jnp
from jax import lax
from jax.experimental import pallas as pl
from jax.experimental.pallas import tpu as pltpu
from jax.experimental.pallas import tpu_sc as plsc

NC = 2
NS = 16
NW = NC * NS
L = 16
NVALS = 256
N = 16 * 512 * 512
PER_W = N // NW
U = 8
NB = 2

CHUNK_A = 32768
NCHUNK_A = PER_W // CHUNK_A
CHUNK_B = 16384
NCHUNK_B = PER_W // CHUNK_B

_mesh = plsc.VectorSubcoreMesh(core_axis_name="c", subcore_axis_name="s")
_params = pltpu.CompilerParams(needs_layout_passes=False)


def _worker_id():
    return lax.axis_index("s") * NC + lax.axis_index("c")


def _presence_body(label_hbm, out_hbm, buf, table, sem0, sem1):
    sems = (sem0, sem1)
    w = _worker_id()
    zeros = jnp.zeros((L,), jnp.int32)
    for i in range(NVALS // L):
        table[pl.ds(i * L, L)] = zeros
    ones = jnp.ones((L,), jnp.int32)
    base = w * jnp.int32(PER_W)

    for b in range(NB):
        pltpu.async_copy(
            label_hbm.at[pl.ds(base + jnp.int32(b * CHUNK_A), CHUNK_A)],
            buf.at[jnp.int32(b)], sems[b])

    def outer(g, _):
        for b in range(NB):
            k = g * jnp.int32(NB) + jnp.int32(b)
            off = base + k * jnp.int32(CHUNK_A)
            pltpu.make_async_copy(
                label_hbm.at[pl.ds(off, CHUNK_A)], buf.at[jnp.int32(b)], sems[b]).wait()

            def inner(i, _, b=b):
                b0 = i * jnp.int32(U * L)
                for u in range(U):
                    idx = buf[b, pl.ds(b0 + jnp.int32(u * L), L)]
                    plsc.store_scatter(table, [idx], ones)
                return 0

            lax.fori_loop(jnp.int32(0), jnp.int32(CHUNK_A // (U * L)),
                          inner, 0)

            nk = k + jnp.int32(NB)

            @pl.when(nk < jnp.int32(NCHUNK_A))
            def _(b=b, nk=nk):
                noff = base + nk * jnp.int32(CHUNK_A)
                pltpu.async_copy(
                    label_hbm.at[pl.ds(noff, CHUNK_A)], buf.at[jnp.int32(b)], sems[b])

        return 0

    lax.fori_loop(jnp.int32(0), jnp.int32(NCHUNK_A // NB), outer, 0)
    pltpu.sync_copy(table, out_hbm.at[jnp.int32(w)])


_presence_call = functools.partial(
    pl.kernel,
    out_type=jax.ShapeDtypeStruct((NW, NVALS), jnp.int32),
    mesh=_mesh,
    compiler_params=_params,
    scratch_types=[
        pltpu.VMEM((NB, CHUNK_A), jnp.int32),
        pltpu.VMEM((NVALS,), jnp.int32),
        pltpu.SemaphoreType.DMA,
        pltpu.SemaphoreType.DMA,
    ],
)(_presence_body)


def _lookup_body(label_hbm, pres_hbm, out_hbm, pbuf, table, lbuf, obuf,
                 isem0, isem1, osem0, osem1):
    isems = (isem0, isem1)
    osems = (osem0, osem1)
    w = _worker_id()
    pltpu.sync_copy(pres_hbm, pbuf)
    base = w * jnp.int32(PER_W)

    for b in range(NB):
        pltpu.async_copy(
            label_hbm.at[pl.ds(base + jnp.int32(b * CHUNK_B), CHUNK_B)],
            lbuf.at[jnp.int32(b)], isems[b])

    carry = jnp.int32(0)
    for ci in range(NVALS // L):
        acc = jnp.zeros((L,), jnp.int32)
        for r in range(NW):
            acc = acc + pbuf[r, pl.ds(ci * L, L)]
        ind = (acc > 0).astype(jnp.int32)
        incl = plsc.cumsum(ind)
        table[pl.ds(ci * L, L)] = (incl - ind) + carry
        carry = carry + jnp.sum(ind, dtype=jnp.int32)

    def outer(g, _):
        for b in range(NB):
            k = g * jnp.int32(NB) + jnp.int32(b)
            off = base + k * jnp.int32(CHUNK_B)
            pltpu.make_async_copy(
                label_hbm.at[pl.ds(off, CHUNK_B)], lbuf.at[jnp.int32(b)],
                isems[b]).wait()

            @pl.when(k >= jnp.int32(NB))
            def _(b=b, off=off):
                pltpu.make_async_copy(
                    obuf.at[jnp.int32(b)], out_hbm.at[pl.ds(off, CHUNK_B)],
                    osems[b]).wait()

            def inner(i, _, b=b):
                b0 = i * jnp.int32(U * L)
                for u in range(U):
                    sl = pl.ds(b0 + jnp.int32(u * L), L)
                    obuf[b, sl] = plsc.load_gather(table, [lbuf[b, sl]])
                return 0

            lax.fori_loop(jnp.int32(0), jnp.int32(CHUNK_B // (U * L)),
                          inner, 0)
            pltpu.async_copy(
                obuf.at[jnp.int32(b)], out_hbm.at[pl.ds(off, CHUNK_B)], osems[b])

            nk = k + jnp.int32(NB)

            @pl.when(nk < jnp.int32(NCHUNK_B))
            def _(b=b, nk=nk):
                noff = base + nk * jnp.int32(CHUNK_B)
                pltpu.async_copy(
                    label_hbm.at[pl.ds(noff, CHUNK_B)], lbuf.at[jnp.int32(b)], isems[b])

        return 0

    lax.fori_loop(jnp.int32(0), jnp.int32(NCHUNK_B // NB), outer, 0)
    for b in range(NB):
        off = base + jnp.int32((NCHUNK_B - NB + b) * CHUNK_B)
        pltpu.make_async_copy(
            obuf.at[jnp.int32(b)], out_hbm.at[pl.ds(off, CHUNK_B)], osems[b]).wait()


_lookup_call = functools.partial(
    pl.kernel,
    out_type=jax.ShapeDtypeStruct((N,), jnp.int32),
    mesh=_mesh,
    compiler_params=_params,
    scratch_types=[
        pltpu.VMEM((NW, NVALS), jnp.int32),
        pltpu.VMEM((NVALS,), jnp.int32),
        pltpu.VMEM((NB, CHUNK_B), jnp.int32),
        pltpu.VMEM((NB, CHUNK_B), jnp.int32),
        pltpu.SemaphoreType.DMA,
        pltpu.SemaphoreType.DMA,
        pltpu.SemaphoreType.DMA,
        pltpu.SemaphoreType.DMA,
    ],
)(_lookup_body)


def kernel(image, label):
    flat = label.reshape(-1).astype(jnp.int32)
    pres = _presence_call(flat)
    out32 = _lookup_call(flat, pres)
    new_label = out32.astype(label.dtype).reshape(label.shape)
    return (image, new_label)


# speedup vs baseline: 1019.3269x; 1.0919x over previous
import functools

import jax
import jax.numpy as jnp
from jax import lax
from jax.experimental import pallas as pl
from jax.experimental.pallas import tpu as pltpu
from jax.experimental.pallas import tpu_sc as plsc

NC = 2
NS = 16
NW = NC * NS
L = 16
NVALS = 256
N = 16 * 512 * 512
W = 512
NROWS = N // W
ROWS_PER_W = NROWS // NW
NB = 2

ROWS_A = 64
NCHUNK_A = ROWS_PER_W // ROWS_A
ROWS_B = 32
NCHUNK_B = ROWS_PER_W // ROWS_B

_mesh = plsc.VectorSubcoreMesh(core_axis_name="c", subcore_axis_name="s")
_params = pltpu.CompilerParams(needs_layout_passes=False)


def _worker_id():
    return lax.axis_index("s") * NC + lax.axis_index("c")


def _presence_body(label_hbm, out_hbm, buf, table, sem0, sem1):
    sems = (sem0, sem1)
    w = _worker_id()
    zeros = jnp.zeros((L,), jnp.int32)
    for i in range(NVALS // L):
        table[pl.ds(i * L, L)] = zeros
    ones = jnp.ones((L,), jnp.int32)
    base = w * jnp.int32(ROWS_PER_W)

    for b in range(NB):
        pltpu.async_copy(
            label_hbm.at[pl.ds(base + jnp.int32(b * ROWS_A), ROWS_A)],
            buf.at[jnp.int32(b)], sems[b])

    def outer(g, _):
        for b in range(NB):
            k = g * jnp.int32(NB) + jnp.int32(b)
            off = base + k * jnp.int32(ROWS_A)
            pltpu.make_async_copy(
                label_hbm.at[pl.ds(off, ROWS_A)], buf.at[jnp.int32(b)],
                sems[b]).wait()

            def inner(r, _, b=b):
                for cc in range(W // L):
                    idx = buf[b, r, pl.ds(jnp.int32(cc * L), L)]
                    plsc.store_scatter(table, [idx], ones)
                return 0

            lax.fori_loop(jnp.int32(0), jnp.int32(ROWS_A), inner, 0)

            nk = k + jnp.int32(NB)

            @pl.when(nk < jnp.int32(NCHUNK_A))
            def _(b=b, nk=nk):
                noff = base + nk * jnp.int32(ROWS_A)
                pltpu.async_copy(
                    label_hbm.at[pl.ds(noff, ROWS_A)], buf.at[jnp.int32(b)],
                    sems[b])

        return 0

    lax.fori_loop(jnp.int32(0), jnp.int32(NCHUNK_A // NB), outer, 0)
    pltpu.sync_copy(table, out_hbm.at[jnp.int32(w)])


_presence_call = functools.partial(
    pl.kernel,
    out_type=jax.ShapeDtypeStruct((NW, NVALS), jnp.int32),
    mesh=_mesh,
    compiler_params=_params,
    scratch_types=[
        pltpu.VMEM((NB, ROWS_A, W), jnp.int32),
        pltpu.VMEM((NVALS,), jnp.int32),
        pltpu.SemaphoreType.DMA,
        pltpu.SemaphoreType.DMA,
    ],
)(_presence_body)


def _lookup_body(label_hbm, pres_hbm, out_hbm, pbuf, table, lbuf, obuf,
                 isem0, isem1, osem0, osem1):
    isems = (isem0, isem1)
    osems = (osem0, osem1)
    w = _worker_id()
    pltpu.sync_copy(pres_hbm, pbuf)
    base = w * jnp.int32(ROWS_PER_W)

    for b in range(NB):
        pltpu.async_copy(
            label_hbm.at[pl.ds(base + jnp.int32(b * ROWS_B), ROWS_B)],
            lbuf.at[jnp.int32(b)], isems[b])

    carry = jnp.int32(0)
    for ci in range(NVALS // L):
        acc = jnp.zeros((L,), jnp.int32)
        for r in range(NW):
            acc = acc + pbuf[r, pl.ds(ci * L, L)]
        ind = (acc > 0).astype(jnp.int32)
        incl = plsc.cumsum(ind)
        table[pl.ds(ci * L, L)] = (incl - ind) + carry
        carry = carry + jnp.sum(ind, dtype=jnp.int32)

    def outer(g, _):
        for b in range(NB):
            k = g * jnp.int32(NB) + jnp.int32(b)
            off = base + k * jnp.int32(ROWS_B)
            pltpu.make_async_copy(
                label_hbm.at[pl.ds(off, ROWS_B)], lbuf.at[jnp.int32(b)],
                isems[b]).wait()

            @pl.when(k >= jnp.int32(NB))
            def _(b=b, off=off):
                pltpu.make_async_copy(
                    obuf.at[jnp.int32(b)], out_hbm.at[pl.ds(off, ROWS_B)],
                    osems[b]).wait()

            def inner(r, _, b=b):
                for cc in range(W // L):
                    sl = pl.ds(jnp.int32(cc * L), L)
                    obuf[b, r, sl] = plsc.load_gather(
                        table, [lbuf[b, r, sl]])
                return 0

            lax.fori_loop(jnp.int32(0), jnp.int32(ROWS_B), inner, 0)
            pltpu.async_copy(
                obuf.at[jnp.int32(b)], out_hbm.at[pl.ds(off, ROWS_B)],
                osems[b])

            nk = k + jnp.int32(NB)

            @pl.when(nk < jnp.int32(NCHUNK_B))
            def _(b=b, nk=nk):
                noff = base + nk * jnp.int32(ROWS_B)
                pltpu.async_copy(
                    label_hbm.at[pl.ds(noff, ROWS_B)], lbuf.at[jnp.int32(b)],
                    isems[b])

        return 0

    lax.fori_loop(jnp.int32(0), jnp.int32(NCHUNK_B // NB), outer, 0)
    for b in range(NB):
        off = base + jnp.int32((NCHUNK_B - NB + b) * ROWS_B)
        pltpu.make_async_copy(
            obuf.at[jnp.int32(b)], out_hbm.at[pl.ds(off, ROWS_B)],
            osems[b]).wait()


_lookup_call = functools.partial(
    pl.kernel,
    out_type=jax.ShapeDtypeStruct((NROWS, W), jnp.int32),
    mesh=_mesh,
    compiler_params=_params,
    scratch_types=[
        pltpu.VMEM((NW, NVALS), jnp.int32),
        pltpu.VMEM((NVALS,), jnp.int32),
        pltpu.VMEM((NB, ROWS_B, W), jnp.int32),
        pltpu.VMEM((NB, ROWS_B, W), jnp.int32),
        pltpu.SemaphoreType.DMA,
        pltpu.SemaphoreType.DMA,
        pltpu.SemaphoreType.DMA,
        pltpu.SemaphoreType.DMA,
    ],
)(_lookup_body)


def kernel(image, label):
    flat = label.astype(jnp.int32).reshape(NROWS, W)
    pres = _presence_call(flat)
    out32 = _lookup_call(flat, pres)
    out_u = lax.bitcast_convert_type(out32, jnp.uint32)
    new_label = out_u.astype(label.dtype).reshape(label.shape)
    return (image, new_label)


# speedup vs baseline: 1159.6338x; 1.1376x over previous
import functools

import jax
import jax.numpy as jnp
from jax import lax
from jax.experimental import pallas as pl
from jax.experimental.pallas import tpu as pltpu
from jax.experimental.pallas import tpu_sc as plsc

NC = 2
NS = 16
NW = NC * NS
L = 16
NVALS = 256
N = 16 * 512 * 512
W = 512
NROWS = N // W
ROWS_PER_W = NROWS // NW
NB = 2

ROWS_A = 64
NCHUNK_A = ROWS_PER_W // ROWS_A
ROWS_B = 32
NCHUNK_B = ROWS_PER_W // ROWS_B

_mesh = plsc.VectorSubcoreMesh(core_axis_name="c", subcore_axis_name="s")
_params = pltpu.CompilerParams(needs_layout_passes=False)


def _worker_id():
    return lax.axis_index("s") * NC + lax.axis_index("c")


def _presence_body(label_hbm, out_hbm, buf, table, sem0, sem1):
    sems = (sem0, sem1)
    w = _worker_id()
    zeros = jnp.zeros((L,), jnp.int32)
    for i in range(NVALS // L):
        table[pl.ds(i * L, L)] = zeros
    ones = jnp.ones((L,), jnp.int32)
    base = w * jnp.int32(ROWS_PER_W)

    for b in range(NB):
        pltpu.async_copy(
            label_hbm.at[pl.ds(base + jnp.int32(b * ROWS_A), ROWS_A)],
            buf.at[jnp.int32(b)], sems[b])

    def outer(g, _):
        for b in range(NB):
            k = g * jnp.int32(NB) + jnp.int32(b)
            off = base + k * jnp.int32(ROWS_A)
            pltpu.make_async_copy(
                label_hbm.at[pl.ds(off, ROWS_A)], buf.at[jnp.int32(b)],
                sems[b]).wait()

            @plsc.parallel_loop(jnp.int32(0), jnp.int32(ROWS_A), step=jnp.int32(1), unroll=2)
            def inner(r, b=b):
                for cc in range(W // L):
                    idx = buf[b, r, pl.ds(jnp.int32(cc * L), L)]
                    plsc.store_scatter(table, [idx], ones)

            nk = k + jnp.int32(NB)

            @pl.when(nk < jnp.int32(NCHUNK_A))
            def _(b=b, nk=nk):
                noff = base + nk * jnp.int32(ROWS_A)
                pltpu.async_copy(
                    label_hbm.at[pl.ds(noff, ROWS_A)], buf.at[jnp.int32(b)],
                    sems[b])

        return 0

    lax.fori_loop(jnp.int32(0), jnp.int32(NCHUNK_A // NB), outer, 0)
    pltpu.sync_copy(table, out_hbm.at[jnp.int32(w)])


_presence_call = functools.partial(
    pl.kernel,
    out_type=jax.ShapeDtypeStruct((NW, NVALS), jnp.int32),
    mesh=_mesh,
    compiler_params=_params,
    scratch_types=[
        pltpu.VMEM((NB, ROWS_A, W), jnp.int32),
        pltpu.VMEM((NVALS,), jnp.int32),
        pltpu.SemaphoreType.DMA,
        pltpu.SemaphoreType.DMA,
    ],
)(_presence_body)


def _lookup_body(label_hbm, pres_hbm, out_hbm, pbuf, table, lbuf, obuf,
                 isem0, isem1, osem0, osem1):
    isems = (isem0, isem1)
    osems = (osem0, osem1)
    w = _worker_id()
    pltpu.sync_copy(pres_hbm, pbuf)
    base = w * jnp.int32(ROWS_PER_W)

    for b in range(NB):
        pltpu.async_copy(
            label_hbm.at[pl.ds(base + jnp.int32(b * ROWS_B), ROWS_B)],
            lbuf.at[jnp.int32(b)], isems[b])

    carry = jnp.int32(0)
    for ci in range(NVALS // L):
        acc = jnp.zeros((L,), jnp.int32)
        for r in range(NW):
            acc = acc + pbuf[r, pl.ds(ci * L, L)]
        ind = (acc > 0).astype(jnp.int32)
        incl = plsc.cumsum(ind)
        table[pl.ds(ci * L, L)] = (incl - ind) + carry
        carry = carry + jnp.sum(ind, dtype=jnp.int32)

    def outer(g, _):
        for b in range(NB):
            k = g * jnp.int32(NB) + jnp.int32(b)
            off = base + k * jnp.int32(ROWS_B)
            pltpu.make_async_copy(
                label_hbm.at[pl.ds(off, ROWS_B)], lbuf.at[jnp.int32(b)],
                isems[b]).wait()

            @pl.when(k >= jnp.int32(NB))
            def _(b=b, off=off):
                pltpu.make_async_copy(
                    obuf.at[jnp.int32(b)], out_hbm.at[pl.ds(off, ROWS_B)],
                    osems[b]).wait()

            @plsc.parallel_loop(jnp.int32(0), jnp.int32(ROWS_B), step=jnp.int32(1), unroll=2)
            def inner(r, b=b):
                for cc in range(W // L):
                    sl = pl.ds(jnp.int32(cc * L), L)
                    obuf[b, r, sl] = plsc.bitcast(
                        plsc.load_gather(table, [lbuf[b, r, sl]]),
                        jnp.uint32)
            pltpu.async_copy(
                obuf.at[jnp.int32(b)], out_hbm.at[pl.ds(off, ROWS_B)],
                osems[b])

            nk = k + jnp.int32(NB)

            @pl.when(nk < jnp.int32(NCHUNK_B))
            def _(b=b, nk=nk):
                noff = base + nk * jnp.int32(ROWS_B)
                pltpu.async_copy(
                    label_hbm.at[pl.ds(noff, ROWS_B)], lbuf.at[jnp.int32(b)],
                    isems[b])

        return 0

    lax.fori_loop(jnp.int32(0), jnp.int32(NCHUNK_B // NB), outer, 0)
    for b in range(NB):
        off = base + jnp.int32((NCHUNK_B - NB + b) * ROWS_B)
        pltpu.make_async_copy(
            obuf.at[jnp.int32(b)], out_hbm.at[pl.ds(off, ROWS_B)],
            osems[b]).wait()


_lookup_call = functools.partial(
    pl.kernel,
    out_type=jax.ShapeDtypeStruct((NROWS, W), jnp.uint32),
    mesh=_mesh,
    compiler_params=_params,
    scratch_types=[
        pltpu.VMEM((NW, NVALS), jnp.int32),
        pltpu.VMEM((NVALS,), jnp.int32),
        pltpu.VMEM((NB, ROWS_B, W), jnp.int32),
        pltpu.VMEM((NB, ROWS_B, W), jnp.uint32),
        pltpu.SemaphoreType.DMA,
        pltpu.SemaphoreType.DMA,
        pltpu.SemaphoreType.DMA,
        pltpu.SemaphoreType.DMA,
    ],
)(_lookup_body)


def kernel(image, label):
    flat = label.astype(jnp.int32).reshape(NROWS, W)
    pres = _presence_call(flat)
    out_u = _lookup_call(flat, pres)
    new_label = out_u.astype(label.dtype).reshape(label.shape)
    return (image, new_label)
